# baseline (speedup 1.0000x reference)
import functools
import jax
import jax.numpy as jnp
from jax import lax
from jax.experimental import pallas as pl
from jax.experimental.pallas import tpu as pltpu

NSAMPLE = 32
EPS = 1e-5
_HI = jax.lax.Precision.HIGHEST


def _mm(a, b):
    return jnp.matmul(a, b, precision=_HI)


def _affine_from_moments(mean_in, M_in, W, b, g, beta):
    my = W @ mean_in + b
    Ey2 = jnp.einsum('oi,ij,oj->o', W, M_in, W, precision=_HI) + 2.0 * b * (W @ mean_in) + b * b
    var = Ey2 - my * my
    scale = g / jnp.sqrt(var + EPS)
    shift = beta - scale * my
    return scale[:, None] * W, scale * b + shift


def _final_bn_kernel(x_ref, g_ref, b_ref, o_ref):
    x = x_ref[...]
    n = x.shape[0] * x.shape[1]
    s = jnp.sum(x, axis=(0, 1))
    s2 = jnp.sum(x * x, axis=(0, 1))
    m = s / n
    v = s2 / n - m * m
    scale = g_ref[0] / jnp.sqrt(v + EPS)
    shift = b_ref[0] - scale * m
    y = jax.nn.relu(x * scale[None, None, :] + shift[None, None, :])
    o_ref[...] = jnp.transpose(y, (0, 2, 1))


def kernel(xyz, points, conv0_w, conv0_b, bn0_g, bn0_b, conv1_w, conv1_b, bn1_g, bn1_b,
           wn0_w, wn0_b, wnbn0_g, wnbn0_b, wn1_w, wn1_b, wnbn1_g, wnbn1_b,
           wn2_w, wn2_b, wnbn2_g, wnbn2_b, lin_w, lin_b, bnl_g, bnl_b):
    K = NSAMPLE
    xyz_t = jnp.transpose(xyz, (0, 2, 1))
    pts = jnp.transpose(points, (0, 2, 1))
    B, N, _ = xyz_t.shape
    P = B * N * K

    dist = -2.0 * jnp.matmul(xyz_t, jnp.transpose(xyz_t, (0, 2, 1)))
    dist = dist + jnp.sum(xyz_t**2, -1)[:, :, None] + jnp.sum(xyz_t**2, -1)[:, None, :]
    _, idx = jax.lax.top_k(-dist, K)

    barange = jnp.arange(B)[:, None, None]
    c = jnp.zeros((B, N), jnp.float32).at[barange, idx].add(1.0)

    X0 = pts.reshape(B * N, 16)
    cw = c.reshape(B * N)
    mean0 = _mm(cw, X0) / P
    M0 = _mm((X0 * cw[:, None]).T, X0) / P
    A1, c1 = _affine_from_moments(mean0, M0, conv0_w, conv0_b, bn0_g, bn0_b)
    X1 = jax.nn.relu(_mm(X0, A1.T) + c1)
    mean1 = _mm(cw, X1) / P
    M1 = _mm((X1 * cw[:, None]).T, X1) / P
    A2, c2 = _affine_from_moments(mean1, M1, conv1_w, conv1_b, bn1_g, bn1_b)
    X2 = jax.nn.relu(_mm(X1, A2.T) + c2)

    nb_xyz = xyz_t[barange, idx]
    G = (nb_xyz - xyz_t[:, :, None, :]).reshape(P, 3)
    meanG = jnp.mean(G, axis=0)
    MG = _mm(G.T, G) / P
    Aw1, cw1 = _affine_from_moments(meanG, MG, wn0_w, wn0_b, wnbn0_g, wnbn0_b)
    U1 = jax.nn.relu(_mm(G, Aw1.T) + cw1)
    meanU1 = jnp.mean(U1, axis=0)
    MU1 = _mm(U1.T, U1) / P
    Aw2, cw2 = _affine_from_moments(meanU1, MU1, wn1_w, wn1_b, wnbn1_g, wnbn1_b)
    U2 = jax.nn.relu(_mm(U1, Aw2.T) + cw2)
    meanU2 = jnp.mean(U2, axis=0)
    MU2 = _mm(U2.T, U2) / P
    Aw3, cw3 = _affine_from_moments(meanU2, MU2, wn2_w, wn2_b, wnbn2_g, wnbn2_b)
    WT = jax.nn.relu(_mm(U2, Aw3.T) + cw3).reshape(B, N, K, 16)

    A = X2.reshape(B, N, 64)[barange, idx]
    m = jnp.einsum('bnkc,bnkj->bncj', A, WT, precision=_HI).reshape(B, N, 1024)
    out_pre = _mm(m, lin_w.T) + lin_b

    out = pl.pallas_call(
        _final_bn_kernel,
        out_shape=jax.ShapeDtypeStruct((B, 64, N), jnp.float32),
    )(out_pre, bnl_g.reshape(1, 64), bnl_b.reshape(1, 64))
    return (xyz, out)


# speedup vs baseline: 5.5824x; 5.5824x over previous
import functools
import jax
import jax.numpy as jnp
from jax import lax
from jax.experimental import pallas as pl
from jax.experimental.pallas import tpu as pltpu
from jax.experimental.pallas import tpu_sc as plsc

NSAMPLE = 32
EPS = 1e-5
_HI = lax.Precision.HIGHEST

_B = 2
_N = 4096
_R = _B * _N
_P = _R * NSAMPLE
_NW = 32
_RPW = _R // _NW
_QT = 512
_SENT = 0x7FFFFFFF


def _dot(a, b, dims):
    return lax.dot_general(a, b, dims, precision=_HI)


def _keys_kernel(xyz_ref, keys_ref, xyzp_ref):
    ii = pl.program_id(1)
    x3 = xyz_ref[0]
    n_all = jnp.sum(x3 * x3, axis=0)
    q3 = xyz_ref[0, :, pl.ds(ii * _QT, _QT)]
    nq = jnp.sum(q3 * q3, axis=0)
    dg = lax.dot_general(q3.astype(jnp.bfloat16), x3.astype(jnp.bfloat16),
                         (((0,), (0,)), ((), ())),
                         preferred_element_type=jnp.float32)
    dist = (-2.0 * dg + nq[:, None]) + n_all[None, :]
    keys_ref[...] = lax.bitcast_convert_type(jnp.maximum(dist, 0.0), jnp.int32)

    @pl.when(ii == 0)
    def _():
        xyzp_ref[...] = jnp.concatenate(
            [x3, jnp.zeros((5, _N), jnp.float32)], axis=0)


def _run_keys(xyz):
    return pl.pallas_call(
        _keys_kernel,
        grid=(_B, _N // _QT),
        in_specs=[pl.BlockSpec((1, 3, _N), lambda b, i: (b, 0, 0))],
        out_specs=[
            pl.BlockSpec((_QT, _N), lambda b, i: (b * (_N // _QT) + i, 0)),
            pl.BlockSpec((8, _N), lambda b, i: (0, b)),
        ],
        out_shape=[
            jax.ShapeDtypeStruct((_R, _N), jnp.int32),
            jax.ShapeDtypeStruct((8, _R), jnp.float32),
        ],
    )(xyz)


def _sc_select_body(keys_hbm, xyzp_hbm, idx_hbm, g_hbm, hist_hbm, macc_hbm,
                    xyzp_v, hist_v, kwin, kbuf, ibuf, ktmp, itmp, macc,
                    gstage, istage):
    wid = lax.axis_index("s") * 2 + lax.axis_index("c")
    base = wid * _RPW
    pltpu.sync_copy(xyzp_hbm.at[pl.ds(0, 3)], xyzp_v)

    zi16 = jnp.zeros((16,), jnp.int32)
    zf16 = jnp.zeros((16,), jnp.float32)
    ones_i = jnp.full((16,), 1, jnp.int32)
    sent_v = jnp.full((16,), _SENT, jnp.int32)
    iota = lax.broadcasted_iota(jnp.int32, (16,), 0)
    k32 = jnp.full((16,), 32, jnp.int32)

    def _zh(i, c):
        hist_v[pl.ds(i * 16, 16)] = zi16
        return c
    lax.fori_loop(0, _R // 16, _zh, 0)
    for ri in range(16):
        macc[ri] = zf16

    def shrink(cur):
        def bit_step(tb, p):
            bit = lax.shift_left(jnp.int32(1), jnp.int32(30) - tb)
            cand = p | jnp.broadcast_to(bit, (16,))
            cnt = zi16
            for v in range(8):
                kv = kbuf[pl.ds(v * 16, 16)]
                cnt = cnt + plsc.all_reduce_population_count(kv < cand)
            return jnp.where(cnt >= k32, p, cand)
        p = lax.fori_loop(0, 31, bit_step, zi16)
        for v in range(8):
            ktmp[pl.ds(v * 16, 16)] = sent_v
        c2 = jnp.int32(0)
        for v in range(8):
            kv = kbuf[pl.ds(v * 16, 16)]
            iv = ibuf[pl.ds(v * 16, 16)]
            m = kv <= p
            mi = jnp.where(m, 1, 0).astype(jnp.int32)
            pos = plsc.cumsum(mi) + jnp.broadcast_to(c2, (16,)) - 1
            plsc.store_scatter(ktmp, [pos], kv, mask=m)
            plsc.store_scatter(itmp, [pos], iv, mask=m)
            c2 = c2 + jnp.sum(mi)
        for v in range(8):
            kbuf[pl.ds(v * 16, 16)] = ktmp[pl.ds(v * 16, 16)]
            ibuf[pl.ds(v * 16, 16)] = itmp[pl.ds(v * 16, 16)]
        return c2, p

    def process_row(r, t):
        pltpu.sync_copy(keys_hbm.at[r], kwin)
        for v in range(8):
            kbuf[pl.ds(v * 16, 16)] = sent_v

        def chunk(i, carry):
            cur, tau = carry
            kv = kwin[pl.ds(i * 16, 16)]
            m = kv < tau
            iv = iota + jnp.broadcast_to(i * 16, (16,))
            mi = jnp.where(m, 1, 0).astype(jnp.int32)
            pos = plsc.cumsum(mi) + jnp.broadcast_to(cur, (16,)) - 1
            plsc.store_scatter(kbuf, [pos], kv, mask=m)
            plsc.store_scatter(ibuf, [pos], iv, mask=m)
            cur = cur + jnp.sum(mi)
            return lax.cond(cur > 96,
                            lambda c: shrink(c),
                            lambda c: (c, tau), cur)

        sent_tau = jnp.broadcast_to(jnp.int32(_SENT), (16,))
        cur, tau = lax.fori_loop(0, _N // 16, chunk, (jnp.int32(0), sent_tau))
        cur, tau = shrink(cur)

        b = lax.shift_right_logical(r, 12)
        jb = jnp.broadcast_to(lax.shift_left(b, 12), (16,))
        c0 = zi16
        c1 = ones_i
        c2v = jnp.full((16,), 2, jnp.int32)
        rv = jnp.broadcast_to(r, (16,))
        xq0 = plsc.load_gather(xyzp_v, [c0, rv])
        xq1 = plsc.load_gather(xyzp_v, [c1, rv])
        xq2 = plsc.load_gather(xyzp_v, [c2v, rv])
        for h in range(2):
            jv = ibuf[pl.ds(h * 16, 16)]
            jg = jv + jb
            gx = plsc.load_gather(xyzp_v, [c0, jg]) - xq0
            gy = plsc.load_gather(xyzp_v, [c1, jg]) - xq1
            gz = plsc.load_gather(xyzp_v, [c2v, jg]) - xq2
            gstage[0, pl.ds(t * 32 + h * 16, 16)] = gx
            gstage[1, pl.ds(t * 32 + h * 16, 16)] = gy
            gstage[2, pl.ds(t * 32 + h * 16, 16)] = gz
            istage[t, pl.ds(h * 16, 16)] = jg
            plsc.addupdate_scatter(hist_v, [jg], ones_i)
            plsc.addupdate(macc.at[0], gx)
            plsc.addupdate(macc.at[1], gy)
            plsc.addupdate(macc.at[2], gz)
            plsc.addupdate(macc.at[3], gx * gx)
            plsc.addupdate(macc.at[4], gx * gy)
            plsc.addupdate(macc.at[5], gx * gz)
            plsc.addupdate(macc.at[6], gy * gy)
            plsc.addupdate(macc.at[7], gy * gz)
            plsc.addupdate(macc.at[8], gz * gz)

    def group(g, c):
        gbase = base + g * 16

        def row(t, c2):
            process_row(gbase + t, t)
            return c2
        lax.fori_loop(0, 16, row, 0)
        pltpu.sync_copy(istage, idx_hbm.at[pl.ds(gbase, 16)])
        for d in range(3):
            pltpu.sync_copy(gstage.at[d], g_hbm.at[d, pl.ds(gbase * 32, 512)])
        return c
    lax.fori_loop(0, _RPW // 16, group, 0)
    pltpu.sync_copy(hist_v, hist_hbm.at[wid])
    pltpu.sync_copy(macc, macc_hbm.at[wid])


def _run_select(keys, xyzp):
    mesh = plsc.VectorSubcoreMesh(core_axis_name="c", subcore_axis_name="s")
    kern = functools.partial(
        pl.kernel, mesh=mesh,
        compiler_params=pltpu.CompilerParams(needs_layout_passes=False),
        out_type=[
            jax.ShapeDtypeStruct((_R, 32), jnp.int32),
            jax.ShapeDtypeStruct((8, _P), jnp.float32),
            jax.ShapeDtypeStruct((_NW, _R), jnp.int32),
            jax.ShapeDtypeStruct((_NW, 16, 16), jnp.float32),
        ],
        scratch_types=[
            pltpu.VMEM((3, _R), jnp.float32),
            pltpu.VMEM((_R,), jnp.int32),
            pltpu.VMEM((_N,), jnp.int32),
            pltpu.VMEM((128,), jnp.int32),
            pltpu.VMEM((128,), jnp.int32),
            pltpu.VMEM((128,), jnp.int32),
            pltpu.VMEM((128,), jnp.int32),
            pltpu.VMEM((16, 16), jnp.float32),
            pltpu.VMEM((4, 512), jnp.float32),
            pltpu.VMEM((16, 32), jnp.int32),
        ],
    )(_sc_select_body)
    return kern(keys, xyzp)


def _sc_gather_body(table_hbm, idxf_hbm, out_hbm, idx_all, rows_v, sem):
    wid = lax.axis_index("s") * 2 + lax.axis_index("c")
    per_w = _P // _NW
    base = wid * per_w
    nwin = per_w // 128
    NB = 4
    pltpu.sync_copy(idxf_hbm.at[pl.ds(base, per_w)], idx_all)

    def _gather(w, slot):
        return pltpu.make_async_copy(
            table_hbm.at[idx_all.at[pl.ds(w * 128, 128)]],
            rows_v.at[slot], sem)
    for s in range(NB):
        _gather(jnp.int32(s), s).start()

    def wgrp(wg, c):
        for s in range(NB):
            w = wg * NB + s
            _gather(w, s).wait()
            pltpu.sync_copy(rows_v.at[s],
                            out_hbm.at[pl.ds(base + w * 128, 128)])
            nw = w + NB

            @pl.when(nw < nwin)
            def _():
                _gather(nw, s).start()
        return c
    lax.fori_loop(0, nwin // NB, wgrp, 0)


def _run_gather(table, idx_flat):
    mesh = plsc.VectorSubcoreMesh(core_axis_name="c", subcore_axis_name="s")
    kern = functools.partial(
        pl.kernel, mesh=mesh,
        compiler_params=pltpu.CompilerParams(needs_layout_passes=False),
        out_type=[jax.ShapeDtypeStruct((_P, 128), jnp.float32)],
        scratch_types=[
            pltpu.VMEM((_P // _NW,), jnp.int32),
            pltpu.VMEM((4, 128, 128), jnp.float32),
            pltpu.SemaphoreType.DMA,
        ],
    )(_sc_gather_body)
    return kern(table, idx_flat)[0]


def _e_kernel(pts_ref, hist_ref, w0_ref, p0_ref, w1_ref, p1_ref,
              x2_ref):
    c = jnp.sum(hist_ref[...].astype(jnp.float32), axis=0)
    t0 = jnp.transpose(pts_ref[0], (1, 0))
    t1 = jnp.transpose(pts_ref[1], (1, 0))
    X0 = jnp.concatenate([t0, t1], axis=0)
    Pf = jnp.float32(_P)

    def fold(X, W, prm, cin, cout):
        bb = prm[0, 0:cout]
        gg = prm[1, 0:cout]
        bt = prm[2, 0:cout]
        mean_in = _dot(c[None, :], X, (((1,), (0,)), ((), ())))[0] / Pf
        M = _dot(X * c[:, None], X, (((0,), (0,)), ((), ()))) / Pf
        my = _dot(W, mean_in[:, None], (((1,), (0,)), ((), ())))[:, 0] + bb
        WM = _dot(W, M, (((1,), (0,)), ((), ())))
        Ey2 = jnp.sum(WM * W, axis=1) + 2.0 * bb * (my - bb) + bb * bb
        var = Ey2 - my * my
        scale = gg * lax.rsqrt(var + EPS)
        Y = lax.dot_general(X.astype(jnp.bfloat16),
                            W.astype(jnp.bfloat16),
                            (((1,), (1,)), ((), ())),
                            preferred_element_type=jnp.float32)
        return jnp.maximum(((Y + bb[None, :]) - my[None, :]) * scale[None, :]
                           + bt[None, :], 0.0)

    X1 = fold(X0, w0_ref[...], p0_ref[...], 16, 32)
    X2 = fold(X1, w1_ref[...], p1_ref[...], 32, 64)
    x2_ref[...] = jnp.concatenate(
        [X2, jnp.zeros((_R, 64), jnp.float32)], axis=1)


def _run_e(points, hist, conv0_w, conv0_b, bn0_g, bn0_b,
           conv1_w, conv1_b, bn1_g, bn1_b):
    w0 = conv0_w
    p0 = jnp.stack([conv0_b, bn0_g, bn0_b], axis=0)
    w1 = conv1_w
    p1 = jnp.stack([conv1_b, bn1_g, bn1_b], axis=0)
    return pl.pallas_call(
        _e_kernel,
        out_shape=jax.ShapeDtypeStruct((_R, 128), jnp.float32),
    )(points, hist, w0, p0, w1, p1)


def _wn_layer(aff, x, cin):
    W = aff[:, 0:cin]
    bb = aff[:, 16][:, None]
    my = aff[:, 17][:, None]
    sc = aff[:, 18][:, None]
    bt = aff[:, 19][:, None]
    Y = lax.dot_general(W.astype(jnp.bfloat16), x.astype(jnp.bfloat16),
                        (((1,), (0,)), ((), ())),
                        preferred_element_type=jnp.float32)
    return jnp.maximum(((Y + bb) - my) * sc + bt, 0.0)


def _km_kernel(nlayer, g_ref, a0_ref, a1_ref, acc_ref):
    st = pl.program_id(0)
    g3 = g_ref[...][0:3, :]
    u = _wn_layer(a0_ref[...], g3, 3)
    if nlayer == 2:
        u = _wn_layer(a1_ref[...], u, 8)

    @pl.when(st == 0)
    def _():
        acc_ref[...] = jnp.zeros_like(acc_ref)

    mu = _dot(u, u, (((1,), (1,)), ((), ())))
    acc_ref[:, 0:8] += mu
    acc_ref[:, 8:9] += jnp.sum(u, axis=1)[:, None]


def _run_km(nlayer, g8, aff0, aff1):
    lt = 16384
    return pl.pallas_call(
        functools.partial(_km_kernel, nlayer),
        grid=(_P // lt,),
        in_specs=[
            pl.BlockSpec((8, lt), lambda i: (0, i)),
            pl.BlockSpec((8, 128), lambda i: (0, 0)),
            pl.BlockSpec((8, 128), lambda i: (0, 0)),
        ],
        out_specs=pl.BlockSpec((8, 128), lambda i: (0, 0)),
        out_shape=jax.ShapeDtypeStruct((8, 128), jnp.float32),
    )(g8, aff0, aff1)


def _kwf_kernel(g_ref, ag_ref, a0_ref, a1_ref, a2_ref, lw_ref, lb_ref,
                out_ref, acc_ref):
    st = pl.program_id(0)
    g3 = g_ref[...][0:3, :]
    u1 = _wn_layer(a0_ref[...], g3, 3)
    u2 = _wn_layer(a1_ref[...], u1, 8)
    wt = _wn_layer(a2_ref[...], u2, 8)
    nt = wt.shape[1] // 32
    wt3 = jnp.transpose(wt, (1, 0)).reshape(nt, 32, 16)
    ag3 = ag_ref[...].reshape(nt, 32, 128)[:, :, 0:64]
    m = lax.dot_general(ag3.astype(jnp.bfloat16), wt3.astype(jnp.bfloat16),
                        (((1,), (1,)), ((0,), (0,))),
                        preferred_element_type=jnp.float32)
    mf = m.reshape(nt, 1024)
    out = lax.dot_general(mf.astype(jnp.bfloat16), lw_ref[...],
                          (((1,), (0,)), ((), ())),
                          preferred_element_type=jnp.float32) \
        + lb_ref[0][None, :]
    out_ref[...] = out

    @pl.when(st == 0)
    def _():
        acc_ref[...] = jnp.zeros_like(acc_ref)

    acc_ref[0:1, 0:64] += jnp.sum(out, axis=0)[None, :]
    acc_ref[1:2, 0:64] += jnp.sum(out * out, axis=0)[None, :]


def _run_kwf(g8, ag, aff0, aff1, aff2, lw_t, lin_b):
    lt = 8192
    nt = lt // 32
    return pl.pallas_call(
        _kwf_kernel,
        grid=(_P // lt,),
        in_specs=[
            pl.BlockSpec((8, lt), lambda i: (0, i)),
            pl.BlockSpec((lt, 128), lambda i: (i, 0)),
            pl.BlockSpec((8, 128), lambda i: (0, 0)),
            pl.BlockSpec((8, 128), lambda i: (0, 0)),
            pl.BlockSpec((16, 128), lambda i: (0, 0)),
            pl.BlockSpec((1024, 64), lambda i: (0, 0)),
            pl.BlockSpec((1, 64), lambda i: (0, 0)),
        ],
        out_specs=[
            pl.BlockSpec((nt, 64), lambda i: (i, 0)),
            pl.BlockSpec((8, 128), lambda i: (0, 0)),
        ],
        out_shape=[
            jax.ShapeDtypeStruct((_R, 64), jnp.float32),
            jax.ShapeDtypeStruct((8, 128), jnp.float32),
        ],
    )(g8, ag, aff0, aff1, aff2, lw_t, lin_b)


def _f_kernel(op_ref, acc_ref, gb_ref, out_ref):
    x = op_ref[...]
    s = acc_ref[0, 0:64]
    s2 = acc_ref[1, 0:64]
    m = s / _R
    v = s2 / _R - m * m
    scale = gb_ref[0, 0:64] * lax.rsqrt(v + EPS)
    shift = gb_ref[1, 0:64] - scale * m
    y = jnp.maximum(x * scale[None, :] + shift[None, :], 0.0)
    y0 = jnp.transpose(y[0:_N], (1, 0))
    y1 = jnp.transpose(y[_N:], (1, 0))
    out_ref[...] = jnp.stack([y0, y1], axis=0)


def _run_f(out_pre, acc, bnl_g, bnl_b):
    gb = jnp.stack([bnl_g, bnl_b], axis=0)
    return pl.pallas_call(
        _f_kernel,
        out_shape=jax.ShapeDtypeStruct((_B, 64, _N), jnp.float32),
    )(out_pre, acc, gb)


def _pack_wn(nrows, mean_in, M_in, W, b, g, beta):
    my = W @ mean_in + b
    Ey2 = jnp.einsum('oi,ij,oj->o', W, M_in, W, precision=_HI) \
        + 2.0 * b * (W @ mean_in) + b * b
    var = Ey2 - my * my
    scale = g / jnp.sqrt(var + EPS)
    cout, cin = W.shape
    aff = jnp.zeros((nrows, 128), jnp.float32)
    aff = aff.at[0:cout, 0:cin].set(W)
    aff = aff.at[0:cout, 16].set(b).at[0:cout, 17].set(my)
    aff = aff.at[0:cout, 18].set(scale).at[0:cout, 19].set(beta)
    return aff


def kernel(xyz, points, conv0_w, conv0_b, bn0_g, bn0_b, conv1_w, conv1_b,
           bn1_g, bn1_b, wn0_w, wn0_b, wnbn0_g, wnbn0_b, wn1_w, wn1_b,
           wnbn1_g, wnbn1_b, wn2_w, wn2_b, wnbn2_g, wnbn2_b,
           lin_w, lin_b, bnl_g, bnl_b):
    keys, xyzp = _run_keys(xyz)
    idx, g8, hist, macc = _run_select(keys, xyzp)

    X2 = _run_e(points, hist, conv0_w, conv0_b, bn0_g, bn0_b,
                conv1_w, conv1_b, bn1_g, bn1_b)

    mac = jnp.sum(macc, axis=0)
    Pf = jnp.float32(_P)
    meanG = jnp.sum(mac[0:3, :], axis=1) / Pf
    xx, xy, xz, yy, yz, zz = [jnp.sum(mac[i]) / Pf for i in range(3, 9)]
    MG = jnp.stack([jnp.stack([xx, xy, xz]),
                    jnp.stack([xy, yy, yz]),
                    jnp.stack([xz, yz, zz])])
    aff0 = _pack_wn(8, meanG, MG, wn0_w, wn0_b, wnbn0_g, wnbn0_b)

    acc1 = _run_km(1, g8, aff0, aff0)
    MU1 = acc1[:, 0:8] / Pf
    mU1 = acc1[:, 8] / Pf
    aff1 = _pack_wn(8, mU1, MU1, wn1_w, wn1_b, wnbn1_g, wnbn1_b)

    acc2 = _run_km(2, g8, aff0, aff1)
    MU2 = acc2[:, 0:8] / Pf
    mU2 = acc2[:, 8] / Pf
    aff2 = _pack_wn(16, mU2, MU2, wn2_w, wn2_b, wnbn2_g, wnbn2_b)

    ag = _run_gather(X2, idx.reshape(_P))

    out_pre, accF = _run_kwf(g8, ag, aff0, aff1, aff2,
                             jnp.transpose(lin_w, (1, 0)).astype(jnp.bfloat16),
                             lin_b[None, :])
    out = _run_f(out_pre, accF, bnl_g, bnl_b)
    return (xyz, out)


# speedup vs baseline: 11.2568x; 2.0165x over previous
import functools
import jax
import jax.numpy as jnp
from jax import lax
from jax.experimental import pallas as pl
from jax.experimental.pallas import tpu as pltpu
from jax.experimental.pallas import tpu_sc as plsc

NSAMPLE = 32
EPS = 1e-5
_HI = lax.Precision.HIGHEST

_B = 2
_N = 4096
_R = _B * _N
_P = _R * NSAMPLE
_NW = 32
_RPW = _R // _NW
_QT = 512
_SENT = 0x7FFFFFFF


def _dot(a, b, dims):
    return lax.dot_general(a, b, dims, precision=_HI)


def _keys_kernel(xyz_ref, keys_ref, xyzp_ref):
    ii = pl.program_id(1)
    x3 = xyz_ref[0]
    n_all = jnp.sum(x3 * x3, axis=0)
    q3 = xyz_ref[0, :, pl.ds(ii * _QT, _QT)]
    nq = jnp.sum(q3 * q3, axis=0)
    dg = lax.dot_general(q3.astype(jnp.bfloat16), x3.astype(jnp.bfloat16),
                         (((0,), (0,)), ((), ())),
                         preferred_element_type=jnp.float32)
    dist = (-2.0 * dg + nq[:, None]) + n_all[None, :]
    keys_ref[...] = lax.bitcast_convert_type(jnp.maximum(dist, 0.0), jnp.int32)

    @pl.when(ii == 0)
    def _():
        xyzp_ref[...] = jnp.concatenate(
            [x3, jnp.zeros((5, _N), jnp.float32)], axis=0)


def _run_keys(xyz):
    return pl.pallas_call(
        _keys_kernel,
        grid=(_B, _N // _QT),
        in_specs=[pl.BlockSpec((1, 3, _N), lambda b, i: (b, 0, 0))],
        out_specs=[
            pl.BlockSpec((_QT, _N), lambda b, i: (b * (_N // _QT) + i, 0)),
            pl.BlockSpec((8, _N), lambda b, i: (0, b)),
        ],
        out_shape=[
            jax.ShapeDtypeStruct((_R, _N), jnp.int32),
            jax.ShapeDtypeStruct((8, _R), jnp.float32),
        ],
    )(xyz)


def _sc_select_body(keys_hbm, xyzp_hbm, idx_hbm, g_hbm, hist_hbm, macc_hbm,
                    xyzp_v, hist_v, kwin0, kwin1, kbuf, ibuf, ktmp, itmp,
                    macc, gstage, istage, sem0, sem1):
    wid = lax.axis_index("s") * 2 + lax.axis_index("c")
    base = wid * _RPW
    pltpu.sync_copy(xyzp_hbm.at[pl.ds(0, 3)], xyzp_v)

    zi16 = jnp.zeros((16,), jnp.int32)
    zf16 = jnp.zeros((16,), jnp.float32)
    ones_i = jnp.full((16,), 1, jnp.int32)
    sent_v = jnp.full((16,), _SENT, jnp.int32)
    iota = lax.broadcasted_iota(jnp.int32, (16,), 0)
    k32 = jnp.full((16,), 32, jnp.int32)
    NV = 12

    def _zh(i, c):
        hist_v[pl.ds(i * 16, 16)] = zi16
        return c
    lax.fori_loop(0, _R // 16, _zh, 0)
    for ri in range(16):
        macc[ri] = zf16

    def shrinkv(op):
        kvs = [kbuf[pl.ds(v * 16, 16)] for v in range(NV)]

        def bit_step(tb, p):
            bit = lax.shift_left(jnp.int32(1), jnp.int32(30) - tb)
            cand = p | jnp.broadcast_to(bit, (16,))
            cnt = zi16
            for v in range(NV):
                cnt = cnt + plsc.all_reduce_population_count(kvs[v] < cand)
            return jnp.where(cnt >= k32, p, cand)
        p = lax.fori_loop(0, 31, bit_step, zi16)
        for v in range(NV):
            ktmp[pl.ds(v * 16, 16)] = sent_v
        c2 = zi16
        for v in range(NV):
            iv = ibuf[pl.ds(v * 16, 16)]
            m = kvs[v] <= p
            mi = jnp.where(m, 1, 0).astype(jnp.int32)
            pos = plsc.cumsum(mi) + c2 - 1
            plsc.store_scatter(ktmp, [pos], kvs[v], mask=m)
            plsc.store_scatter(itmp, [pos], iv, mask=m)
            c2 = c2 + plsc.all_reduce_population_count(m)
        for v in range(NV):
            kbuf[pl.ds(v * 16, 16)] = ktmp[pl.ds(v * 16, 16)]
            ibuf[pl.ds(v * 16, 16)] = itmp[pl.ds(v * 16, 16)]
        return c2, p

    def process_row(kref, r, t):
        for v in range(NV):
            kbuf[pl.ds(v * 16, 16)] = sent_v

        def blk(ci, carry):
            cur_v, tau = carry
            kvs, ms, pcs = [], [], []
            tot = zi16
            for q in range(4):
                kv = kref[pl.ds(ci * 64 + q * 16, 16)]
                m = kv < tau
                pc = plsc.all_reduce_population_count(m)
                kvs.append(kv)
                ms.append(m)
                pcs.append(pc)
                tot = tot + pc

            def do_append(op):
                cur2, tau2 = op
                for q in range(4):
                    mi = jnp.where(ms[q], 1, 0).astype(jnp.int32)
                    pos = plsc.cumsum(mi) + cur2 - 1
                    plsc.store_scatter(kbuf, [pos], kvs[q], mask=ms[q])
                    iv = iota + jnp.broadcast_to(ci * 64 + q * 16, (16,))
                    plsc.store_scatter(ibuf, [pos], iv, mask=ms[q])
                    cur2 = cur2 + pcs[q]
                return lax.cond(cur2[0] > 128, shrinkv, lambda o: o,
                                (cur2, tau2))
            return lax.cond(tot[0] > 0, do_append, lambda o: o, (cur_v, tau))

        sent_tau = jnp.broadcast_to(jnp.int32(_SENT), (16,))
        cur_v, tau = lax.fori_loop(0, _N // 64, blk, (zi16, sent_tau))
        cur_v, tau = shrinkv((cur_v, tau))

        b = lax.shift_right_logical(r, 12)
        jb = jnp.broadcast_to(lax.shift_left(b, 12), (16,))
        c0 = zi16
        c1 = ones_i
        c2v = jnp.full((16,), 2, jnp.int32)
        rv = jnp.broadcast_to(r, (16,))
        xq0 = plsc.load_gather(xyzp_v, [c0, rv])
        xq1 = plsc.load_gather(xyzp_v, [c1, rv])
        xq2 = plsc.load_gather(xyzp_v, [c2v, rv])
        for h in range(2):
            jv = ibuf[pl.ds(h * 16, 16)]
            jg = jv + jb
            gx = plsc.load_gather(xyzp_v, [c0, jg]) - xq0
            gy = plsc.load_gather(xyzp_v, [c1, jg]) - xq1
            gz = plsc.load_gather(xyzp_v, [c2v, jg]) - xq2
            gstage[0, pl.ds(t * 32 + h * 16, 16)] = gx
            gstage[1, pl.ds(t * 32 + h * 16, 16)] = gy
            gstage[2, pl.ds(t * 32 + h * 16, 16)] = gz
            istage[t, pl.ds(h * 16, 16)] = jg
            plsc.addupdate_scatter(hist_v, [jg], ones_i)
            plsc.addupdate(macc.at[0], gx)
            plsc.addupdate(macc.at[1], gy)
            plsc.addupdate(macc.at[2], gz)
            plsc.addupdate(macc.at[3], gx * gx)
            plsc.addupdate(macc.at[4], gx * gy)
            plsc.addupdate(macc.at[5], gx * gz)
            plsc.addupdate(macc.at[6], gy * gy)
            plsc.addupdate(macc.at[7], gy * gz)
            plsc.addupdate(macc.at[8], gz * gz)

    def start_row(r, kref, sem):
        return pltpu.make_async_copy(keys_hbm.at[r], kref, sem)

    start_row(base, kwin0, sem0).start()

    def group(g, c):
        gbase = base + g * 16

        def pair(u, c2):
            r0 = gbase + 2 * u
            start_row(r0, kwin0, sem0).wait()
            start_row(r0 + 1, kwin1, sem1).start()
            process_row(kwin0, r0, 2 * u)
            start_row(r0 + 1, kwin1, sem1).wait()

            @pl.when(r0 + 2 <= base + _RPW - 1)
            def _():
                start_row(r0 + 2, kwin0, sem0).start()
            process_row(kwin1, r0 + 1, 2 * u + 1)
            return c2
        lax.fori_loop(0, 8, pair, 0)
        pltpu.sync_copy(istage, idx_hbm.at[pl.ds(gbase, 16)])
        for d in range(3):
            pltpu.sync_copy(gstage.at[d], g_hbm.at[d, pl.ds(gbase * 32, 512)])
        return c
    lax.fori_loop(0, _RPW // 16, group, 0)
    pltpu.sync_copy(hist_v, hist_hbm.at[wid])
    pltpu.sync_copy(macc, macc_hbm.at[wid])


def _run_select(keys, xyzp):
    mesh = plsc.VectorSubcoreMesh(core_axis_name="c", subcore_axis_name="s")
    kern = functools.partial(
        pl.kernel, mesh=mesh,
        compiler_params=pltpu.CompilerParams(needs_layout_passes=False),
        out_type=[
            jax.ShapeDtypeStruct((_R, 32), jnp.int32),
            jax.ShapeDtypeStruct((8, _P), jnp.float32),
            jax.ShapeDtypeStruct((_NW, _R), jnp.int32),
            jax.ShapeDtypeStruct((_NW, 16, 16), jnp.float32),
        ],
        scratch_types=[
            pltpu.VMEM((3, _R), jnp.float32),
            pltpu.VMEM((_R,), jnp.int32),
            pltpu.VMEM((_N,), jnp.int32),
            pltpu.VMEM((_N,), jnp.int32),
            pltpu.VMEM((192,), jnp.int32),
            pltpu.VMEM((192,), jnp.int32),
            pltpu.VMEM((192,), jnp.int32),
            pltpu.VMEM((192,), jnp.int32),
            pltpu.VMEM((16, 16), jnp.float32),
            pltpu.VMEM((4, 512), jnp.float32),
            pltpu.VMEM((16, 32), jnp.int32),
            pltpu.SemaphoreType.DMA,
            pltpu.SemaphoreType.DMA,
        ],
    )(_sc_select_body)
    return kern(keys, xyzp)


def _sc_gather_body(table_hbm, idxf_hbm, out_hbm, idx_all, rows_v, sem):
    wid = lax.axis_index("s") * 2 + lax.axis_index("c")
    per_w = _P // _NW
    base = wid * per_w
    nwin = per_w // 128
    NB = 4
    pltpu.sync_copy(idxf_hbm.at[pl.ds(base, per_w)], idx_all)

    def _gather(w, slot):
        return pltpu.make_async_copy(
            table_hbm.at[idx_all.at[pl.ds(w * 128, 128)]],
            rows_v.at[slot], sem)
    for s in range(NB):
        _gather(jnp.int32(s), s).start()

    def wgrp(wg, c):
        for s in range(NB):
            w = wg * NB + s
            _gather(w, s).wait()
            pltpu.sync_copy(rows_v.at[s],
                            out_hbm.at[pl.ds(base + w * 128, 128)])
            nw = w + NB

            @pl.when(nw < nwin)
            def _():
                _gather(nw, s).start()
        return c
    lax.fori_loop(0, nwin // NB, wgrp, 0)


def _run_gather(table, idx_flat):
    mesh = plsc.VectorSubcoreMesh(core_axis_name="c", subcore_axis_name="s")
    kern = functools.partial(
        pl.kernel, mesh=mesh,
        compiler_params=pltpu.CompilerParams(needs_layout_passes=False),
        out_type=[jax.ShapeDtypeStruct((_P, 128), jnp.float32)],
        scratch_types=[
            pltpu.VMEM((_P // _NW,), jnp.int32),
            pltpu.VMEM((4, 128, 128), jnp.float32),
            pltpu.SemaphoreType.DMA,
        ],
    )(_sc_gather_body)
    return kern(table, idx_flat)[0]


def _e_kernel(pts_ref, hist_ref, w0_ref, p0_ref, w1_ref, p1_ref,
              x2_ref):
    c = jnp.sum(hist_ref[...].astype(jnp.float32), axis=0)
    t0 = jnp.transpose(pts_ref[0], (1, 0))
    t1 = jnp.transpose(pts_ref[1], (1, 0))
    X0 = jnp.concatenate([t0, t1], axis=0)
    Pf = jnp.float32(_P)

    def fold(X, W, prm, cin, cout):
        bb = prm[0, 0:cout]
        gg = prm[1, 0:cout]
        bt = prm[2, 0:cout]
        mean_in = _dot(c[None, :], X, (((1,), (0,)), ((), ())))[0] / Pf
        M = _dot(X * c[:, None], X, (((0,), (0,)), ((), ()))) / Pf
        my = _dot(W, mean_in[:, None], (((1,), (0,)), ((), ())))[:, 0] + bb
        WM = _dot(W, M, (((1,), (0,)), ((), ())))
        Ey2 = jnp.sum(WM * W, axis=1) + 2.0 * bb * (my - bb) + bb * bb
        var = Ey2 - my * my
        scale = gg * lax.rsqrt(var + EPS)
        Y = lax.dot_general(X.astype(jnp.bfloat16),
                            W.astype(jnp.bfloat16),
                            (((1,), (1,)), ((), ())),
                            preferred_element_type=jnp.float32)
        return jnp.maximum(((Y + bb[None, :]) - my[None, :]) * scale[None, :]
                           + bt[None, :], 0.0)

    X1 = fold(X0, w0_ref[...], p0_ref[...], 16, 32)
    X2 = fold(X1, w1_ref[...], p1_ref[...], 32, 64)
    x2_ref[...] = jnp.concatenate(
        [X2, jnp.zeros((_R, 64), jnp.float32)], axis=1)


def _run_e(points, hist, conv0_w, conv0_b, bn0_g, bn0_b,
           conv1_w, conv1_b, bn1_g, bn1_b):
    w0 = conv0_w
    p0 = jnp.stack([conv0_b, bn0_g, bn0_b], axis=0)
    w1 = conv1_w
    p1 = jnp.stack([conv1_b, bn1_g, bn1_b], axis=0)
    return pl.pallas_call(
        _e_kernel,
        out_shape=jax.ShapeDtypeStruct((_R, 128), jnp.float32),
    )(points, hist, w0, p0, w1, p1)


def _wn_layer(aff, x, cin):
    W = aff[:, 0:cin]
    bb = aff[:, 16][:, None]
    my = aff[:, 17][:, None]
    sc = aff[:, 18][:, None]
    bt = aff[:, 19][:, None]
    Y = lax.dot_general(W.astype(jnp.bfloat16), x.astype(jnp.bfloat16),
                        (((1,), (0,)), ((), ())),
                        preferred_element_type=jnp.float32)
    return jnp.maximum(((Y + bb) - my) * sc + bt, 0.0)


def _km_kernel(nlayer, g_ref, a0_ref, a1_ref, acc_ref):
    st = pl.program_id(0)
    g3 = g_ref[...][0:3, :]
    u = _wn_layer(a0_ref[...], g3, 3)
    if nlayer == 2:
        u = _wn_layer(a1_ref[...], u, 8)

    @pl.when(st == 0)
    def _():
        acc_ref[...] = jnp.zeros_like(acc_ref)

    mu = _dot(u, u, (((1,), (1,)), ((), ())))
    acc_ref[:, 0:8] += mu
    acc_ref[:, 8:9] += jnp.sum(u, axis=1)[:, None]


def _run_km(nlayer, g8, aff0, aff1):
    lt = 16384
    return pl.pallas_call(
        functools.partial(_km_kernel, nlayer),
        grid=(_P // lt,),
        in_specs=[
            pl.BlockSpec((8, lt), lambda i: (0, i)),
            pl.BlockSpec((8, 128), lambda i: (0, 0)),
            pl.BlockSpec((8, 128), lambda i: (0, 0)),
        ],
        out_specs=pl.BlockSpec((8, 128), lambda i: (0, 0)),
        out_shape=jax.ShapeDtypeStruct((8, 128), jnp.float32),
    )(g8, aff0, aff1)


def _kwf_kernel(g_ref, ag_ref, a0_ref, a1_ref, a2_ref, lw_ref, lb_ref,
                out_ref, acc_ref):
    st = pl.program_id(0)
    g3 = g_ref[...][0:3, :]
    u1 = _wn_layer(a0_ref[...], g3, 3)
    u2 = _wn_layer(a1_ref[...], u1, 8)
    wt = _wn_layer(a2_ref[...], u2, 8)
    nt = wt.shape[1] // 32
    wt3 = jnp.transpose(wt, (1, 0)).reshape(nt, 32, 16)
    ag3 = ag_ref[...].reshape(nt, 32, 128)[:, :, 0:64]
    m = lax.dot_general(ag3.astype(jnp.bfloat16), wt3.astype(jnp.bfloat16),
                        (((1,), (1,)), ((0,), (0,))),
                        preferred_element_type=jnp.float32)
    mf = m.reshape(nt, 1024)
    out = lax.dot_general(mf.astype(jnp.bfloat16), lw_ref[...],
                          (((1,), (0,)), ((), ())),
                          preferred_element_type=jnp.float32) \
        + lb_ref[0][None, :]
    out_ref[...] = out

    @pl.when(st == 0)
    def _():
        acc_ref[...] = jnp.zeros_like(acc_ref)

    acc_ref[0:1, 0:64] += jnp.sum(out, axis=0)[None, :]
    acc_ref[1:2, 0:64] += jnp.sum(out * out, axis=0)[None, :]


def _run_kwf(g8, ag, aff0, aff1, aff2, lw_t, lin_b):
    lt = 8192
    nt = lt // 32
    return pl.pallas_call(
        _kwf_kernel,
        grid=(_P // lt,),
        in_specs=[
            pl.BlockSpec((8, lt), lambda i: (0, i)),
            pl.BlockSpec((lt, 128), lambda i: (i, 0)),
            pl.BlockSpec((8, 128), lambda i: (0, 0)),
            pl.BlockSpec((8, 128), lambda i: (0, 0)),
            pl.BlockSpec((16, 128), lambda i: (0, 0)),
            pl.BlockSpec((1024, 64), lambda i: (0, 0)),
            pl.BlockSpec((1, 64), lambda i: (0, 0)),
        ],
        out_specs=[
            pl.BlockSpec((nt, 64), lambda i: (i, 0)),
            pl.BlockSpec((8, 128), lambda i: (0, 0)),
        ],
        out_shape=[
            jax.ShapeDtypeStruct((_R, 64), jnp.float32),
            jax.ShapeDtypeStruct((8, 128), jnp.float32),
        ],
    )(g8, ag, aff0, aff1, aff2, lw_t, lin_b)


def _f_kernel(op_ref, acc_ref, gb_ref, out_ref):
    x = op_ref[...]
    s = acc_ref[0, 0:64]
    s2 = acc_ref[1, 0:64]
    m = s / _R
    v = s2 / _R - m * m
    scale = gb_ref[0, 0:64] * lax.rsqrt(v + EPS)
    shift = gb_ref[1, 0:64] - scale * m
    y = jnp.maximum(x * scale[None, :] + shift[None, :], 0.0)
    y0 = jnp.transpose(y[0:_N], (1, 0))
    y1 = jnp.transpose(y[_N:], (1, 0))
    out_ref[...] = jnp.stack([y0, y1], axis=0)


def _run_f(out_pre, acc, bnl_g, bnl_b):
    gb = jnp.stack([bnl_g, bnl_b], axis=0)
    return pl.pallas_call(
        _f_kernel,
        out_shape=jax.ShapeDtypeStruct((_B, 64, _N), jnp.float32),
    )(out_pre, acc, gb)


def _pack_wn(nrows, mean_in, M_in, W, b, g, beta):
    my = W @ mean_in + b
    Ey2 = jnp.einsum('oi,ij,oj->o', W, M_in, W, precision=_HI) \
        + 2.0 * b * (W @ mean_in) + b * b
    var = Ey2 - my * my
    scale = g / jnp.sqrt(var + EPS)
    cout, cin = W.shape
    aff = jnp.zeros((nrows, 128), jnp.float32)
    aff = aff.at[0:cout, 0:cin].set(W)
    aff = aff.at[0:cout, 16].set(b).at[0:cout, 17].set(my)
    aff = aff.at[0:cout, 18].set(scale).at[0:cout, 19].set(beta)
    return aff


def kernel(xyz, points, conv0_w, conv0_b, bn0_g, bn0_b, conv1_w, conv1_b,
           bn1_g, bn1_b, wn0_w, wn0_b, wnbn0_g, wnbn0_b, wn1_w, wn1_b,
           wnbn1_g, wnbn1_b, wn2_w, wn2_b, wnbn2_g, wnbn2_b,
           lin_w, lin_b, bnl_g, bnl_b):
    keys, xyzp = _run_keys(xyz)
    idx, g8, hist, macc = _run_select(keys, xyzp)

    X2 = _run_e(points, hist, conv0_w, conv0_b, bn0_g, bn0_b,
                conv1_w, conv1_b, bn1_g, bn1_b)

    mac = jnp.sum(macc, axis=0)
    Pf = jnp.float32(_P)
    meanG = jnp.sum(mac[0:3, :], axis=1) / Pf
    xx, xy, xz, yy, yz, zz = [jnp.sum(mac[i]) / Pf for i in range(3, 9)]
    MG = jnp.stack([jnp.stack([xx, xy, xz]),
                    jnp.stack([xy, yy, yz]),
                    jnp.stack([xz, yz, zz])])
    aff0 = _pack_wn(8, meanG, MG, wn0_w, wn0_b, wnbn0_g, wnbn0_b)

    acc1 = _run_km(1, g8, aff0, aff0)
    MU1 = acc1[:, 0:8] / Pf
    mU1 = acc1[:, 8] / Pf
    aff1 = _pack_wn(8, mU1, MU1, wn1_w, wn1_b, wnbn1_g, wnbn1_b)

    acc2 = _run_km(2, g8, aff0, aff1)
    MU2 = acc2[:, 0:8] / Pf
    mU2 = acc2[:, 8] / Pf
    aff2 = _pack_wn(16, mU2, MU2, wn2_w, wn2_b, wnbn2_g, wnbn2_b)

    ag = _run_gather(X2, idx.reshape(_P))

    out_pre, accF = _run_kwf(g8, ag, aff0, aff1, aff2,
                             jnp.transpose(lin_w, (1, 0)).astype(jnp.bfloat16),
                             lin_b[None, :])
    out = _run_f(out_pre, accF, bnl_g, bnl_b)
    return (xyz, out)


# speedup vs baseline: 11.4772x; 1.0196x over previous
import functools
import jax
import jax.numpy as jnp
from jax import lax
from jax.experimental import pallas as pl
from jax.experimental.pallas import tpu as pltpu
from jax.experimental.pallas import tpu_sc as plsc

NSAMPLE = 32
EPS = 1e-5
_HI = lax.Precision.HIGHEST

_B = 2
_N = 4096
_R = _B * _N
_P = _R * NSAMPLE
_NW = 32
_RPW = _R // _NW
_QT = 512
_SENT = 0x7FFFFFFF


def _dot(a, b, dims):
    return lax.dot_general(a, b, dims, precision=_HI)


def _keys_kernel(xyz_ref, keys_ref, xyzp_ref):
    ii = pl.program_id(1)
    x3 = xyz_ref[0]
    n_all = jnp.sum(x3 * x3, axis=0)
    q3 = xyz_ref[0, :, pl.ds(ii * _QT, _QT)]
    nq = jnp.sum(q3 * q3, axis=0)
    dg = lax.dot_general(q3.astype(jnp.bfloat16), x3.astype(jnp.bfloat16),
                         (((0,), (0,)), ((), ())),
                         preferred_element_type=jnp.float32)
    dist = (-2.0 * dg + nq[:, None]) + n_all[None, :]
    keys_ref[...] = lax.bitcast_convert_type(jnp.maximum(dist, 0.0), jnp.int32)

    @pl.when(ii == 0)
    def _():
        xyzp_ref[...] = jnp.concatenate(
            [x3, jnp.zeros((5, _N), jnp.float32)], axis=0)


def _run_keys(xyz):
    return pl.pallas_call(
        _keys_kernel,
        grid=(_B, _N // _QT),
        in_specs=[pl.BlockSpec((1, 3, _N), lambda b, i: (b, 0, 0))],
        out_specs=[
            pl.BlockSpec((_QT, _N), lambda b, i: (b * (_N // _QT) + i, 0)),
            pl.BlockSpec((8, _N), lambda b, i: (0, b)),
        ],
        out_shape=[
            jax.ShapeDtypeStruct((_R, _N), jnp.int32),
            jax.ShapeDtypeStruct((8, _R), jnp.float32),
        ],
    )(xyz)


def _sc_select_body(keys_hbm, xyzp_hbm, idx_hbm, g_hbm, hist_hbm, macc_hbm,
                    xyzp_v, hist_v, kwin0, kwin1, kbuf, ibuf, ktmp, itmp,
                    macc, gstage, istage, sem0, sem1):
    wid = lax.axis_index("s") * 2 + lax.axis_index("c")
    base = wid * _RPW
    pltpu.sync_copy(xyzp_hbm.at[pl.ds(0, 3)], xyzp_v)

    zi16 = jnp.zeros((16,), jnp.int32)
    zf16 = jnp.zeros((16,), jnp.float32)
    ones_i = jnp.full((16,), 1, jnp.int32)
    sent_v = jnp.full((16,), _SENT, jnp.int32)
    iota = lax.broadcasted_iota(jnp.int32, (16,), 0)
    k32 = jnp.full((16,), 32, jnp.int32)
    NV = 12

    def _zh(i, c):
        hist_v[pl.ds(i * 16, 16)] = zi16
        return c
    lax.fori_loop(0, _R // 16, _zh, 0)
    for ri in range(16):
        macc[ri] = zf16

    def shrinkv(op):
        kvs = [kbuf[pl.ds(v * 16, 16)] for v in range(NV)]

        def bit_step(tb, p):
            bit = lax.shift_left(jnp.int32(1), jnp.int32(30) - tb)
            cand = p | jnp.broadcast_to(bit, (16,))
            cnt = zi16
            for v in range(NV):
                cnt = cnt + plsc.all_reduce_population_count(kvs[v] < cand)
            return jnp.where(cnt >= k32, p, cand)
        p = lax.fori_loop(0, 31, bit_step, zi16)
        for v in range(NV):
            ktmp[pl.ds(v * 16, 16)] = sent_v
        c2 = zi16
        for v in range(NV):
            iv = ibuf[pl.ds(v * 16, 16)]
            m = kvs[v] <= p
            mi = jnp.where(m, 1, 0).astype(jnp.int32)
            pos = plsc.cumsum(mi) + c2 - 1
            plsc.store_scatter(ktmp, [pos], kvs[v], mask=m)
            plsc.store_scatter(itmp, [pos], iv, mask=m)
            c2 = c2 + plsc.all_reduce_population_count(m)
        for v in range(NV):
            kbuf[pl.ds(v * 16, 16)] = ktmp[pl.ds(v * 16, 16)]
            ibuf[pl.ds(v * 16, 16)] = itmp[pl.ds(v * 16, 16)]
        return c2, p

    def process_row(kref, r, t):
        for v in range(NV):
            kbuf[pl.ds(v * 16, 16)] = sent_v

        def blk(ci, carry):
            cur_v, tau = carry
            kvs, ms, pcs = [], [], []
            tot = zi16
            for q in range(4):
                kv = kref[pl.ds(ci * 64 + q * 16, 16)]
                m = kv < tau
                pc = plsc.all_reduce_population_count(m)
                kvs.append(kv)
                ms.append(m)
                pcs.append(pc)
                tot = tot + pc

            def do_append(op):
                cur2, tau2 = op
                off = cur2
                for q in range(4):
                    mi = jnp.where(ms[q], 1, 0).astype(jnp.int32)
                    pos = plsc.cumsum(mi) + off - 1
                    plsc.store_scatter(kbuf, [pos], kvs[q], mask=ms[q])
                    iv = iota + jnp.broadcast_to(ci * 64 + q * 16, (16,))
                    plsc.store_scatter(ibuf, [pos], iv, mask=ms[q])
                    off = off + pcs[q]
                return lax.cond(off[0] > 128, shrinkv, lambda o: o,
                                (off, tau2))
            return lax.cond(tot[0] > 0, do_append, lambda o: o, (cur_v, tau))

        for v in range(NV):
            kbuf[pl.ds(v * 16, 16)] = kref[pl.ds(v * 16, 16)]
            ibuf[pl.ds(v * 16, 16)] = iota + jnp.broadcast_to(
                jnp.int32(v * 16), (16,))
        cur_v, tau = shrinkv((zi16, zi16))
        cur_v, tau = lax.fori_loop(NV * 16 // 64, _N // 64, blk, (cur_v, tau))
        cur_v, tau = shrinkv((cur_v, tau))

        b = lax.shift_right_logical(r, 12)
        jb = jnp.broadcast_to(lax.shift_left(b, 12), (16,))
        c0 = zi16
        c1 = ones_i
        c2v = jnp.full((16,), 2, jnp.int32)
        rv = jnp.broadcast_to(r, (16,))
        xq0 = plsc.load_gather(xyzp_v, [c0, rv])
        xq1 = plsc.load_gather(xyzp_v, [c1, rv])
        xq2 = plsc.load_gather(xyzp_v, [c2v, rv])
        for h in range(2):
            jv = ibuf[pl.ds(h * 16, 16)]
            jg = jv + jb
            gx = plsc.load_gather(xyzp_v, [c0, jg]) - xq0
            gy = plsc.load_gather(xyzp_v, [c1, jg]) - xq1
            gz = plsc.load_gather(xyzp_v, [c2v, jg]) - xq2
            gstage[0, pl.ds(t * 32 + h * 16, 16)] = gx
            gstage[1, pl.ds(t * 32 + h * 16, 16)] = gy
            gstage[2, pl.ds(t * 32 + h * 16, 16)] = gz
            istage[t, pl.ds(h * 16, 16)] = jg
            plsc.addupdate_scatter(hist_v, [jg], ones_i)
            plsc.addupdate(macc.at[0], gx)
            plsc.addupdate(macc.at[1], gy)
            plsc.addupdate(macc.at[2], gz)
            plsc.addupdate(macc.at[3], gx * gx)
            plsc.addupdate(macc.at[4], gx * gy)
            plsc.addupdate(macc.at[5], gx * gz)
            plsc.addupdate(macc.at[6], gy * gy)
            plsc.addupdate(macc.at[7], gy * gz)
            plsc.addupdate(macc.at[8], gz * gz)

    def start_row(r, kref, sem):
        return pltpu.make_async_copy(keys_hbm.at[r], kref, sem)

    start_row(base, kwin0, sem0).start()

    def group(g, c):
        gbase = base + g * 16

        def pair(u, c2):
            r0 = gbase + 2 * u
            start_row(r0, kwin0, sem0).wait()
            start_row(r0 + 1, kwin1, sem1).start()
            process_row(kwin0, r0, 2 * u)
            start_row(r0 + 1, kwin1, sem1).wait()

            @pl.when(r0 + 2 <= base + _RPW - 1)
            def _():
                start_row(r0 + 2, kwin0, sem0).start()
            process_row(kwin1, r0 + 1, 2 * u + 1)
            return c2
        lax.fori_loop(0, 8, pair, 0)
        pltpu.sync_copy(istage, idx_hbm.at[pl.ds(gbase, 16)])
        for d in range(3):
            pltpu.sync_copy(gstage.at[d], g_hbm.at[d, pl.ds(gbase * 32, 512)])
        return c
    lax.fori_loop(0, _RPW // 16, group, 0)
    pltpu.sync_copy(hist_v, hist_hbm.at[wid])
    pltpu.sync_copy(macc, macc_hbm.at[wid])


def _run_select(keys, xyzp):
    mesh = plsc.VectorSubcoreMesh(core_axis_name="c", subcore_axis_name="s")
    kern = functools.partial(
        pl.kernel, mesh=mesh,
        compiler_params=pltpu.CompilerParams(needs_layout_passes=False),
        out_type=[
            jax.ShapeDtypeStruct((_R, 32), jnp.int32),
            jax.ShapeDtypeStruct((8, _P), jnp.float32),
            jax.ShapeDtypeStruct((_NW, _R), jnp.int32),
            jax.ShapeDtypeStruct((_NW, 16, 16), jnp.float32),
        ],
        scratch_types=[
            pltpu.VMEM((3, _R), jnp.float32),
            pltpu.VMEM((_R,), jnp.int32),
            pltpu.VMEM((_N,), jnp.int32),
            pltpu.VMEM((_N,), jnp.int32),
            pltpu.VMEM((192,), jnp.int32),
            pltpu.VMEM((192,), jnp.int32),
            pltpu.VMEM((192,), jnp.int32),
            pltpu.VMEM((192,), jnp.int32),
            pltpu.VMEM((16, 16), jnp.float32),
            pltpu.VMEM((4, 512), jnp.float32),
            pltpu.VMEM((16, 32), jnp.int32),
            pltpu.SemaphoreType.DMA,
            pltpu.SemaphoreType.DMA,
        ],
    )(_sc_select_body)
    return kern(keys, xyzp)


def _sc_gather_body(table_hbm, idxf_hbm, out_hbm, idx_all, rows_v, sem):
    wid = lax.axis_index("s") * 2 + lax.axis_index("c")
    per_w = _P // _NW
    base = wid * per_w
    nwin = per_w // 128
    NB = 4
    pltpu.sync_copy(idxf_hbm.at[pl.ds(base, per_w)], idx_all)

    def _gather(w, slot):
        return pltpu.make_async_copy(
            table_hbm.at[idx_all.at[pl.ds(w * 128, 128)]],
            rows_v.at[slot], sem)
    for s in range(NB):
        _gather(jnp.int32(s), s).start()

    def wgrp(wg, c):
        for s in range(NB):
            w = wg * NB + s
            _gather(w, s).wait()
            pltpu.sync_copy(rows_v.at[s],
                            out_hbm.at[pl.ds(base + w * 128, 128)])
            nw = w + NB

            @pl.when(nw < nwin)
            def _():
                _gather(nw, s).start()
        return c
    lax.fori_loop(0, nwin // NB, wgrp, 0)


def _run_gather(table, idx_flat):
    mesh = plsc.VectorSubcoreMesh(core_axis_name="c", subcore_axis_name="s")
    kern = functools.partial(
        pl.kernel, mesh=mesh,
        compiler_params=pltpu.CompilerParams(needs_layout_passes=False),
        out_type=[jax.ShapeDtypeStruct((_P, 128), jnp.float32)],
        scratch_types=[
            pltpu.VMEM((_P // _NW,), jnp.int32),
            pltpu.VMEM((4, 128, 128), jnp.float32),
            pltpu.SemaphoreType.DMA,
        ],
    )(_sc_gather_body)
    return kern(table, idx_flat)[0]


def _e_kernel(pts_ref, hist_ref, w0_ref, p0_ref, w1_ref, p1_ref,
              x2_ref):
    c = jnp.sum(hist_ref[...].astype(jnp.float32), axis=0)
    t0 = jnp.transpose(pts_ref[0], (1, 0))
    t1 = jnp.transpose(pts_ref[1], (1, 0))
    X0 = jnp.concatenate([t0, t1], axis=0)
    Pf = jnp.float32(_P)

    def fold(X, W, prm, cin, cout):
        bb = prm[0, 0:cout]
        gg = prm[1, 0:cout]
        bt = prm[2, 0:cout]
        mean_in = _dot(c[None, :], X, (((1,), (0,)), ((), ())))[0] / Pf
        M = _dot(X * c[:, None], X, (((0,), (0,)), ((), ()))) / Pf
        my = _dot(W, mean_in[:, None], (((1,), (0,)), ((), ())))[:, 0] + bb
        WM = _dot(W, M, (((1,), (0,)), ((), ())))
        Ey2 = jnp.sum(WM * W, axis=1) + 2.0 * bb * (my - bb) + bb * bb
        var = Ey2 - my * my
        scale = gg * lax.rsqrt(var + EPS)
        Y = lax.dot_general(X.astype(jnp.bfloat16),
                            W.astype(jnp.bfloat16),
                            (((1,), (1,)), ((), ())),
                            preferred_element_type=jnp.float32)
        return jnp.maximum(((Y + bb[None, :]) - my[None, :]) * scale[None, :]
                           + bt[None, :], 0.0)

    X1 = fold(X0, w0_ref[...], p0_ref[...], 16, 32)
    X2 = fold(X1, w1_ref[...], p1_ref[...], 32, 64)
    x2_ref[...] = jnp.concatenate(
        [X2, jnp.zeros((_R, 64), jnp.float32)], axis=1)


def _run_e(points, hist, conv0_w, conv0_b, bn0_g, bn0_b,
           conv1_w, conv1_b, bn1_g, bn1_b):
    w0 = conv0_w
    p0 = jnp.stack([conv0_b, bn0_g, bn0_b], axis=0)
    w1 = conv1_w
    p1 = jnp.stack([conv1_b, bn1_g, bn1_b], axis=0)
    return pl.pallas_call(
        _e_kernel,
        out_shape=jax.ShapeDtypeStruct((_R, 128), jnp.float32),
    )(points, hist, w0, p0, w1, p1)


def _wn_layer(aff, x, cin):
    W = aff[:, 0:cin]
    bb = aff[:, 16][:, None]
    my = aff[:, 17][:, None]
    sc = aff[:, 18][:, None]
    bt = aff[:, 19][:, None]
    Y = lax.dot_general(W.astype(jnp.bfloat16), x.astype(jnp.bfloat16),
                        (((1,), (0,)), ((), ())),
                        preferred_element_type=jnp.float32)
    return jnp.maximum(((Y + bb) - my) * sc + bt, 0.0)


def _km_kernel(nlayer, g_ref, a0_ref, a1_ref, acc_ref):
    st = pl.program_id(0)
    g3 = g_ref[...][0:3, :]
    u = _wn_layer(a0_ref[...], g3, 3)
    if nlayer == 2:
        u = _wn_layer(a1_ref[...], u, 8)

    @pl.when(st == 0)
    def _():
        acc_ref[...] = jnp.zeros_like(acc_ref)

    mu = _dot(u, u, (((1,), (1,)), ((), ())))
    acc_ref[:, 0:8] += mu
    acc_ref[:, 8:9] += jnp.sum(u, axis=1)[:, None]


def _run_km(nlayer, g8, aff0, aff1):
    lt = 16384
    return pl.pallas_call(
        functools.partial(_km_kernel, nlayer),
        grid=(_P // lt,),
        in_specs=[
            pl.BlockSpec((8, lt), lambda i: (0, i)),
            pl.BlockSpec((8, 128), lambda i: (0, 0)),
            pl.BlockSpec((8, 128), lambda i: (0, 0)),
        ],
        out_specs=pl.BlockSpec((8, 128), lambda i: (0, 0)),
        out_shape=jax.ShapeDtypeStruct((8, 128), jnp.float32),
    )(g8, aff0, aff1)


def _kwf_kernel(g_ref, ag_ref, a0_ref, a1_ref, a2_ref, lw_ref, lb_ref,
                out_ref, acc_ref):
    st = pl.program_id(0)
    g3 = g_ref[...][0:3, :]
    u1 = _wn_layer(a0_ref[...], g3, 3)
    u2 = _wn_layer(a1_ref[...], u1, 8)
    wt = _wn_layer(a2_ref[...], u2, 8)
    nt = wt.shape[1] // 32
    wt3 = jnp.transpose(wt, (1, 0)).reshape(nt, 32, 16)
    ag3 = ag_ref[...].reshape(nt, 32, 128)[:, :, 0:64]
    m = lax.dot_general(ag3.astype(jnp.bfloat16), wt3.astype(jnp.bfloat16),
                        (((1,), (1,)), ((0,), (0,))),
                        preferred_element_type=jnp.float32)
    mf = m.reshape(nt, 1024)
    out = lax.dot_general(mf.astype(jnp.bfloat16), lw_ref[...],
                          (((1,), (0,)), ((), ())),
                          preferred_element_type=jnp.float32) \
        + lb_ref[0][None, :]
    out_ref[...] = out

    @pl.when(st == 0)
    def _():
        acc_ref[...] = jnp.zeros_like(acc_ref)

    acc_ref[0:1, 0:64] += jnp.sum(out, axis=0)[None, :]
    acc_ref[1:2, 0:64] += jnp.sum(out * out, axis=0)[None, :]


def _run_kwf(g8, ag, aff0, aff1, aff2, lw_t, lin_b):
    lt = 8192
    nt = lt // 32
    return pl.pallas_call(
        _kwf_kernel,
        grid=(_P // lt,),
        in_specs=[
            pl.BlockSpec((8, lt), lambda i: (0, i)),
            pl.BlockSpec((lt, 128), lambda i: (i, 0)),
            pl.BlockSpec((8, 128), lambda i: (0, 0)),
            pl.BlockSpec((8, 128), lambda i: (0, 0)),
            pl.BlockSpec((16, 128), lambda i: (0, 0)),
            pl.BlockSpec((1024, 64), lambda i: (0, 0)),
            pl.BlockSpec((1, 64), lambda i: (0, 0)),
        ],
        out_specs=[
            pl.BlockSpec((nt, 64), lambda i: (i, 0)),
            pl.BlockSpec((8, 128), lambda i: (0, 0)),
        ],
        out_shape=[
            jax.ShapeDtypeStruct((_R, 64), jnp.float32),
            jax.ShapeDtypeStruct((8, 128), jnp.float32),
        ],
    )(g8, ag, aff0, aff1, aff2, lw_t, lin_b)


def _f_kernel(op_ref, acc_ref, gb_ref, out_ref):
    x = op_ref[...]
    s = acc_ref[0, 0:64]
    s2 = acc_ref[1, 0:64]
    m = s / _R
    v = s2 / _R - m * m
    scale = gb_ref[0, 0:64] * lax.rsqrt(v + EPS)
    shift = gb_ref[1, 0:64] - scale * m
    y = jnp.maximum(x * scale[None, :] + shift[None, :], 0.0)
    y0 = jnp.transpose(y[0:_N], (1, 0))
    y1 = jnp.transpose(y[_N:], (1, 0))
    out_ref[...] = jnp.stack([y0, y1], axis=0)


def _run_f(out_pre, acc, bnl_g, bnl_b):
    gb = jnp.stack([bnl_g, bnl_b], axis=0)
    return pl.pallas_call(
        _f_kernel,
        out_shape=jax.ShapeDtypeStruct((_B, 64, _N), jnp.float32),
    )(out_pre, acc, gb)


def _pack_wn(nrows, mean_in, M_in, W, b, g, beta):
    my = W @ mean_in + b
    Ey2 = jnp.einsum('oi,ij,oj->o', W, M_in, W, precision=_HI) \
        + 2.0 * b * (W @ mean_in) + b * b
    var = Ey2 - my * my
    scale = g / jnp.sqrt(var + EPS)
    cout, cin = W.shape
    aff = jnp.zeros((nrows, 128), jnp.float32)
    aff = aff.at[0:cout, 0:cin].set(W)
    aff = aff.at[0:cout, 16].set(b).at[0:cout, 17].set(my)
    aff = aff.at[0:cout, 18].set(scale).at[0:cout, 19].set(beta)
    return aff


def kernel(xyz, points, conv0_w, conv0_b, bn0_g, bn0_b, conv1_w, conv1_b,
           bn1_g, bn1_b, wn0_w, wn0_b, wnbn0_g, wnbn0_b, wn1_w, wn1_b,
           wnbn1_g, wnbn1_b, wn2_w, wn2_b, wnbn2_g, wnbn2_b,
           lin_w, lin_b, bnl_g, bnl_b):
    keys, xyzp = _run_keys(xyz)
    idx, g8, hist, macc = _run_select(keys, xyzp)

    X2 = _run_e(points, hist, conv0_w, conv0_b, bn0_g, bn0_b,
                conv1_w, conv1_b, bn1_g, bn1_b)

    mac = jnp.sum(macc, axis=0)
    Pf = jnp.float32(_P)
    meanG = jnp.sum(mac[0:3, :], axis=1) / Pf
    xx, xy, xz, yy, yz, zz = [jnp.sum(mac[i]) / Pf for i in range(3, 9)]
    MG = jnp.stack([jnp.stack([xx, xy, xz]),
                    jnp.stack([xy, yy, yz]),
                    jnp.stack([xz, yz, zz])])
    aff0 = _pack_wn(8, meanG, MG, wn0_w, wn0_b, wnbn0_g, wnbn0_b)

    acc1 = _run_km(1, g8, aff0, aff0)
    MU1 = acc1[:, 0:8] / Pf
    mU1 = acc1[:, 8] / Pf
    aff1 = _pack_wn(8, mU1, MU1, wn1_w, wn1_b, wnbn1_g, wnbn1_b)

    acc2 = _run_km(2, g8, aff0, aff1)
    MU2 = acc2[:, 0:8] / Pf
    mU2 = acc2[:, 8] / Pf
    aff2 = _pack_wn(16, mU2, MU2, wn2_w, wn2_b, wnbn2_g, wnbn2_b)

    ag = _run_gather(X2, idx.reshape(_P))

    out_pre, accF = _run_kwf(g8, ag, aff0, aff1, aff2,
                             jnp.transpose(lin_w, (1, 0)).astype(jnp.bfloat16),
                             lin_b[None, :])
    out = _run_f(out_pre, accF, bnl_g, bnl_b)
    return (xyz, out)


# speedup vs baseline: 13.2474x; 1.1542x over previous
import functools
import jax
import jax.numpy as jnp
from jax import lax
from jax.experimental import pallas as pl
from jax.experimental.pallas import tpu as pltpu
from jax.experimental.pallas import tpu_sc as plsc

NSAMPLE = 32
EPS = 1e-5
_HI = lax.Precision.HIGHEST

_B = 2
_N = 4096
_R = _B * _N
_P = _R * NSAMPLE
_NW = 32
_RPW = _R // _NW
_QT = 512
_SENT = 0x7FFFFFFF


def _dot(a, b, dims):
    return lax.dot_general(a, b, dims, precision=_HI)


def _keys_kernel(xyz_ref, keys_ref, xyzp_ref):
    ii = pl.program_id(1)
    x3 = xyz_ref[0]
    n_all = jnp.sum(x3 * x3, axis=0)
    q3 = xyz_ref[0, :, pl.ds(ii * _QT, _QT)]
    nq = jnp.sum(q3 * q3, axis=0)
    dg = lax.dot_general(q3.astype(jnp.bfloat16), x3.astype(jnp.bfloat16),
                         (((0,), (0,)), ((), ())),
                         preferred_element_type=jnp.float32)
    dist = (-2.0 * dg + nq[:, None]) + n_all[None, :]
    keys_ref[...] = lax.bitcast_convert_type(jnp.maximum(dist, 0.0), jnp.int32)

    @pl.when(ii == 0)
    def _():
        xyzp_ref[...] = jnp.concatenate(
            [x3, jnp.zeros((5, _N), jnp.float32)], axis=0)


def _run_keys(xyz):
    return pl.pallas_call(
        _keys_kernel,
        grid=(_B, _N // _QT),
        in_specs=[pl.BlockSpec((1, 3, _N), lambda b, i: (b, 0, 0))],
        out_specs=[
            pl.BlockSpec((_QT, _N), lambda b, i: (b * (_N // _QT) + i, 0)),
            pl.BlockSpec((8, _N), lambda b, i: (0, b)),
        ],
        out_shape=[
            jax.ShapeDtypeStruct((_R, _N), jnp.int32),
            jax.ShapeDtypeStruct((8, _R), jnp.float32),
        ],
    )(xyz)


def _sc_select_body(keys_hbm, xyzp_hbm, idx_hbm, g_hbm, hist_hbm, macc_hbm,
                    xyzp_v, hist_v, kwin0, kwin1, kbuf, ibuf, ktmp, itmp,
                    macc, gstage, istage, sem0, sem1):
    wid = lax.axis_index("s") * 2 + lax.axis_index("c")
    base = wid * _RPW
    pltpu.sync_copy(xyzp_hbm.at[pl.ds(0, 3)], xyzp_v)

    zi16 = jnp.zeros((16,), jnp.int32)
    zf16 = jnp.zeros((16,), jnp.float32)
    ones_i = jnp.full((16,), 1, jnp.int32)
    sent_v = jnp.full((16,), _SENT, jnp.int32)
    iota = lax.broadcasted_iota(jnp.int32, (16,), 0)
    k32 = jnp.full((16,), 32, jnp.int32)
    NV = 16

    def _zh(i, c):
        hist_v[pl.ds(i * 16, 16)] = zi16
        return c
    lax.fori_loop(0, _R // 16, _zh, 0)
    for ri in range(16):
        macc[ri] = zf16

    def shrinkv(op):
        kvs = [kbuf[pl.ds(v * 16, 16)] for v in range(NV)]

        def bit_step(tb, p):
            bit = lax.shift_left(jnp.int32(1), jnp.int32(30) - tb)
            cand = p | jnp.broadcast_to(bit, (16,))
            cnt = zi16
            for v in range(NV):
                cnt = cnt + plsc.all_reduce_population_count(kvs[v] < cand)
            return jnp.where(cnt >= k32, p, cand)
        p = lax.fori_loop(0, 31, bit_step, zi16)
        for v in range(NV):
            ktmp[pl.ds(v * 16, 16)] = sent_v
        c2 = zi16
        for v in range(NV):
            iv = ibuf[pl.ds(v * 16, 16)]
            m = kvs[v] <= p
            mi = jnp.where(m, 1, 0).astype(jnp.int32)
            pos = plsc.cumsum(mi) + c2 - 1
            plsc.store_scatter(ktmp, [pos], kvs[v], mask=m)
            plsc.store_scatter(itmp, [pos], iv, mask=m)
            c2 = c2 + plsc.all_reduce_population_count(m)
        for v in range(NV):
            kbuf[pl.ds(v * 16, 16)] = ktmp[pl.ds(v * 16, 16)]
            ibuf[pl.ds(v * 16, 16)] = itmp[pl.ds(v * 16, 16)]
        return c2, p

    def process_row(kref, r, t):
        for v in range(NV):
            kbuf[pl.ds(v * 16, 16)] = sent_v

        def blk(ci, carry):
            cur_v, tau = carry
            kvs, ms, pcs = [], [], []
            tot = zi16
            for q in range(8):
                kv = kref[pl.ds(ci * 128 + q * 16, 16)]
                m = kv < tau
                pc = plsc.all_reduce_population_count(m)
                kvs.append(kv)
                ms.append(m)
                pcs.append(pc)
                tot = tot + pc

            def do_append(op):
                cur2, tau2 = op
                off = cur2
                for q in range(8):
                    mi = jnp.where(ms[q], 1, 0).astype(jnp.int32)
                    pos = plsc.cumsum(mi) + off - 1
                    plsc.store_scatter(kbuf, [pos], kvs[q], mask=ms[q])
                    iv = iota + jnp.broadcast_to(ci * 128 + q * 16, (16,))
                    plsc.store_scatter(ibuf, [pos], iv, mask=ms[q])
                    off = off + pcs[q]
                return lax.cond(off[0] > 128, shrinkv, lambda o: o,
                                (off, tau2))
            return lax.cond(tot[0] > 0, do_append, lambda o: o, (cur_v, tau))

        for v in range(NV):
            kbuf[pl.ds(v * 16, 16)] = kref[pl.ds(v * 16, 16)]
            ibuf[pl.ds(v * 16, 16)] = iota + jnp.broadcast_to(
                jnp.int32(v * 16), (16,))
        cur_v, tau = shrinkv((zi16, zi16))
        cur_v, tau = lax.fori_loop(NV * 16 // 128, _N // 128, blk, (cur_v, tau))
        cur_v, tau = shrinkv((cur_v, tau))

        b = lax.shift_right_logical(r, 12)
        jb = jnp.broadcast_to(lax.shift_left(b, 12), (16,))
        c0 = zi16
        c1 = ones_i
        c2v = jnp.full((16,), 2, jnp.int32)
        rv = jnp.broadcast_to(r, (16,))
        xq0 = plsc.load_gather(xyzp_v, [c0, rv])
        xq1 = plsc.load_gather(xyzp_v, [c1, rv])
        xq2 = plsc.load_gather(xyzp_v, [c2v, rv])
        for h in range(2):
            jv = ibuf[pl.ds(h * 16, 16)]
            jg = jv + jb
            gx = plsc.load_gather(xyzp_v, [c0, jg]) - xq0
            gy = plsc.load_gather(xyzp_v, [c1, jg]) - xq1
            gz = plsc.load_gather(xyzp_v, [c2v, jg]) - xq2
            gstage[0, pl.ds(t * 32 + h * 16, 16)] = gx
            gstage[1, pl.ds(t * 32 + h * 16, 16)] = gy
            gstage[2, pl.ds(t * 32 + h * 16, 16)] = gz
            istage[t, pl.ds(h * 16, 16)] = jg
            plsc.addupdate_scatter(hist_v, [jg], ones_i)
            plsc.addupdate(macc.at[0], gx)
            plsc.addupdate(macc.at[1], gy)
            plsc.addupdate(macc.at[2], gz)
            plsc.addupdate(macc.at[3], gx * gx)
            plsc.addupdate(macc.at[4], gx * gy)
            plsc.addupdate(macc.at[5], gx * gz)
            plsc.addupdate(macc.at[6], gy * gy)
            plsc.addupdate(macc.at[7], gy * gz)
            plsc.addupdate(macc.at[8], gz * gz)

    def start_row(r, kref, sem):
        return pltpu.make_async_copy(keys_hbm.at[r], kref, sem)

    start_row(base, kwin0, sem0).start()

    def group(g, c):
        gbase = base + g * 16

        def pair(u, c2):
            r0 = gbase + 2 * u
            start_row(r0, kwin0, sem0).wait()
            start_row(r0 + 1, kwin1, sem1).start()
            process_row(kwin0, r0, 2 * u)
            start_row(r0 + 1, kwin1, sem1).wait()

            @pl.when(r0 + 2 <= base + _RPW - 1)
            def _():
                start_row(r0 + 2, kwin0, sem0).start()
            process_row(kwin1, r0 + 1, 2 * u + 1)
            return c2
        lax.fori_loop(0, 8, pair, 0)
        pltpu.sync_copy(istage, idx_hbm.at[pl.ds(gbase, 16)])
        for d in range(3):
            pltpu.sync_copy(gstage.at[d], g_hbm.at[d, pl.ds(gbase * 32, 512)])
        return c
    lax.fori_loop(0, _RPW // 16, group, 0)
    pltpu.sync_copy(hist_v, hist_hbm.at[wid])
    pltpu.sync_copy(macc, macc_hbm.at[wid])


def _run_select(keys, xyzp):
    mesh = plsc.VectorSubcoreMesh(core_axis_name="c", subcore_axis_name="s")
    kern = functools.partial(
        pl.kernel, mesh=mesh,
        compiler_params=pltpu.CompilerParams(needs_layout_passes=False),
        out_type=[
            jax.ShapeDtypeStruct((_R, 32), jnp.int32),
            jax.ShapeDtypeStruct((8, _P), jnp.float32),
            jax.ShapeDtypeStruct((_NW, _R), jnp.int32),
            jax.ShapeDtypeStruct((_NW, 16, 16), jnp.float32),
        ],
        scratch_types=[
            pltpu.VMEM((3, _R), jnp.float32),
            pltpu.VMEM((_R,), jnp.int32),
            pltpu.VMEM((_N,), jnp.int32),
            pltpu.VMEM((_N,), jnp.int32),
            pltpu.VMEM((256,), jnp.int32),
            pltpu.VMEM((256,), jnp.int32),
            pltpu.VMEM((256,), jnp.int32),
            pltpu.VMEM((256,), jnp.int32),
            pltpu.VMEM((16, 16), jnp.float32),
            pltpu.VMEM((4, 512), jnp.float32),
            pltpu.VMEM((16, 32), jnp.int32),
            pltpu.SemaphoreType.DMA,
            pltpu.SemaphoreType.DMA,
        ],
    )(_sc_select_body)
    return kern(keys, xyzp)


def _sc_gather_body(table_hbm, idxf_hbm, out_hbm, idx_all, rows_v, sem):
    wid = lax.axis_index("s") * 2 + lax.axis_index("c")
    per_w = _P // _NW
    base = wid * per_w
    nwin = per_w // 128
    NB = 4
    pltpu.sync_copy(idxf_hbm.at[pl.ds(base, per_w)], idx_all)

    def _gather(w, slot):
        return pltpu.make_async_copy(
            table_hbm.at[idx_all.at[pl.ds(w * 128, 128)]],
            rows_v.at[slot], sem)
    for s in range(NB):
        _gather(jnp.int32(s), s).start()

    def wgrp(wg, c):
        for s in range(NB):
            w = wg * NB + s
            _gather(w, s).wait()
            pltpu.sync_copy(rows_v.at[s],
                            out_hbm.at[pl.ds(base + w * 128, 128)])
            nw = w + NB

            @pl.when(nw < nwin)
            def _():
                _gather(nw, s).start()
        return c
    lax.fori_loop(0, nwin // NB, wgrp, 0)


def _run_gather(table, idx_flat):
    mesh = plsc.VectorSubcoreMesh(core_axis_name="c", subcore_axis_name="s")
    kern = functools.partial(
        pl.kernel, mesh=mesh,
        compiler_params=pltpu.CompilerParams(needs_layout_passes=False),
        out_type=[jax.ShapeDtypeStruct((_P, 128), jnp.float32)],
        scratch_types=[
            pltpu.VMEM((_P // _NW,), jnp.int32),
            pltpu.VMEM((4, 128, 128), jnp.float32),
            pltpu.SemaphoreType.DMA,
        ],
    )(_sc_gather_body)
    return kern(table, idx_flat)[0]


def _e_kernel(pts_ref, hist_ref, w0_ref, p0_ref, w1_ref, p1_ref,
              x2_ref):
    c = jnp.sum(hist_ref[...].astype(jnp.float32), axis=0)
    t0 = jnp.transpose(pts_ref[0], (1, 0))
    t1 = jnp.transpose(pts_ref[1], (1, 0))
    X0 = jnp.concatenate([t0, t1], axis=0)
    Pf = jnp.float32(_P)

    def fold(X, W, prm, cin, cout):
        bb = prm[0, 0:cout]
        gg = prm[1, 0:cout]
        bt = prm[2, 0:cout]
        mean_in = _dot(c[None, :], X, (((1,), (0,)), ((), ())))[0] / Pf
        M = _dot(X * c[:, None], X, (((0,), (0,)), ((), ()))) / Pf
        my = _dot(W, mean_in[:, None], (((1,), (0,)), ((), ())))[:, 0] + bb
        WM = _dot(W, M, (((1,), (0,)), ((), ())))
        Ey2 = jnp.sum(WM * W, axis=1) + 2.0 * bb * (my - bb) + bb * bb
        var = Ey2 - my * my
        scale = gg * lax.rsqrt(var + EPS)
        Y = lax.dot_general(X.astype(jnp.bfloat16),
                            W.astype(jnp.bfloat16),
                            (((1,), (1,)), ((), ())),
                            preferred_element_type=jnp.float32)
        return jnp.maximum(((Y + bb[None, :]) - my[None, :]) * scale[None, :]
                           + bt[None, :], 0.0)

    X1 = fold(X0, w0_ref[...], p0_ref[...], 16, 32)
    X2 = fold(X1, w1_ref[...], p1_ref[...], 32, 64)
    x2_ref[...] = jnp.concatenate(
        [X2, jnp.zeros((_R, 64), jnp.float32)], axis=1)


def _run_e(points, hist, conv0_w, conv0_b, bn0_g, bn0_b,
           conv1_w, conv1_b, bn1_g, bn1_b):
    w0 = conv0_w
    p0 = jnp.stack([conv0_b, bn0_g, bn0_b], axis=0)
    w1 = conv1_w
    p1 = jnp.stack([conv1_b, bn1_g, bn1_b], axis=0)
    return pl.pallas_call(
        _e_kernel,
        out_shape=jax.ShapeDtypeStruct((_R, 128), jnp.float32),
    )(points, hist, w0, p0, w1, p1)


def _wn_layer(aff, x, cin):
    W = aff[:, 0:cin]
    bb = aff[:, 16][:, None]
    my = aff[:, 17][:, None]
    sc = aff[:, 18][:, None]
    bt = aff[:, 19][:, None]
    Y = lax.dot_general(W.astype(jnp.bfloat16), x.astype(jnp.bfloat16),
                        (((1,), (0,)), ((), ())),
                        preferred_element_type=jnp.float32)
    return jnp.maximum(((Y + bb) - my) * sc + bt, 0.0)


def _km_kernel(nlayer, g_ref, a0_ref, a1_ref, acc_ref):
    st = pl.program_id(0)
    g3 = g_ref[...][0:3, :]
    u = _wn_layer(a0_ref[...], g3, 3)
    if nlayer == 2:
        u = _wn_layer(a1_ref[...], u, 8)

    @pl.when(st == 0)
    def _():
        acc_ref[...] = jnp.zeros_like(acc_ref)

    mu = _dot(u, u, (((1,), (1,)), ((), ())))
    acc_ref[:, 0:8] += mu
    acc_ref[:, 8:9] += jnp.sum(u, axis=1)[:, None]


def _run_km(nlayer, g8, aff0, aff1):
    lt = 16384
    return pl.pallas_call(
        functools.partial(_km_kernel, nlayer),
        grid=(_P // lt,),
        in_specs=[
            pl.BlockSpec((8, lt), lambda i: (0, i)),
            pl.BlockSpec((8, 128), lambda i: (0, 0)),
            pl.BlockSpec((8, 128), lambda i: (0, 0)),
        ],
        out_specs=pl.BlockSpec((8, 128), lambda i: (0, 0)),
        out_shape=jax.ShapeDtypeStruct((8, 128), jnp.float32),
    )(g8, aff0, aff1)


def _kwf_kernel(g_ref, ag_ref, a0_ref, a1_ref, a2_ref, lw_ref, lb_ref,
                out_ref, acc_ref):
    st = pl.program_id(0)
    g3 = g_ref[...][0:3, :]
    u1 = _wn_layer(a0_ref[...], g3, 3)
    u2 = _wn_layer(a1_ref[...], u1, 8)
    wt = _wn_layer(a2_ref[...], u2, 8)
    nt = wt.shape[1] // 32
    wt3 = jnp.transpose(wt, (1, 0)).reshape(nt, 32, 16)
    ag3 = ag_ref[...].reshape(nt, 32, 128)[:, :, 0:64]
    m = lax.dot_general(ag3.astype(jnp.bfloat16), wt3.astype(jnp.bfloat16),
                        (((1,), (1,)), ((0,), (0,))),
                        preferred_element_type=jnp.float32)
    mf = m.reshape(nt, 1024)
    out = lax.dot_general(mf.astype(jnp.bfloat16), lw_ref[...],
                          (((1,), (0,)), ((), ())),
                          preferred_element_type=jnp.float32) \
        + lb_ref[0][None, :]
    out_ref[...] = out

    @pl.when(st == 0)
    def _():
        acc_ref[...] = jnp.zeros_like(acc_ref)

    acc_ref[0:1, 0:64] += jnp.sum(out, axis=0)[None, :]
    acc_ref[1:2, 0:64] += jnp.sum(out * out, axis=0)[None, :]


def _run_kwf(g8, ag, aff0, aff1, aff2, lw_t, lin_b):
    lt = 8192
    nt = lt // 32
    return pl.pallas_call(
        _kwf_kernel,
        grid=(_P // lt,),
        in_specs=[
            pl.BlockSpec((8, lt), lambda i: (0, i)),
            pl.BlockSpec((lt, 128), lambda i: (i, 0)),
            pl.BlockSpec((8, 128), lambda i: (0, 0)),
            pl.BlockSpec((8, 128), lambda i: (0, 0)),
            pl.BlockSpec((16, 128), lambda i: (0, 0)),
            pl.BlockSpec((1024, 64), lambda i: (0, 0)),
            pl.BlockSpec((1, 64), lambda i: (0, 0)),
        ],
        out_specs=[
            pl.BlockSpec((nt, 64), lambda i: (i, 0)),
            pl.BlockSpec((8, 128), lambda i: (0, 0)),
        ],
        out_shape=[
            jax.ShapeDtypeStruct((_R, 64), jnp.float32),
            jax.ShapeDtypeStruct((8, 128), jnp.float32),
        ],
    )(g8, ag, aff0, aff1, aff2, lw_t, lin_b)


def _f_kernel(op_ref, acc_ref, gb_ref, out_ref):
    x = op_ref[...]
    s = acc_ref[0, 0:64]
    s2 = acc_ref[1, 0:64]
    m = s / _R
    v = s2 / _R - m * m
    scale = gb_ref[0, 0:64] * lax.rsqrt(v + EPS)
    shift = gb_ref[1, 0:64] - scale * m
    y = jnp.maximum(x * scale[None, :] + shift[None, :], 0.0)
    y0 = jnp.transpose(y[0:_N], (1, 0))
    y1 = jnp.transpose(y[_N:], (1, 0))
    out_ref[...] = jnp.stack([y0, y1], axis=0)


def _run_f(out_pre, acc, bnl_g, bnl_b):
    gb = jnp.stack([bnl_g, bnl_b], axis=0)
    return pl.pallas_call(
        _f_kernel,
        out_shape=jax.ShapeDtypeStruct((_B, 64, _N), jnp.float32),
    )(out_pre, acc, gb)


def _pack_wn(nrows, mean_in, M_in, W, b, g, beta):
    my = W @ mean_in + b
    Ey2 = jnp.einsum('oi,ij,oj->o', W, M_in, W, precision=_HI) \
        + 2.0 * b * (W @ mean_in) + b * b
    var = Ey2 - my * my
    scale = g / jnp.sqrt(var + EPS)
    cout, cin = W.shape
    aff = jnp.zeros((nrows, 128), jnp.float32)
    aff = aff.at[0:cout, 0:cin].set(W)
    aff = aff.at[0:cout, 16].set(b).at[0:cout, 17].set(my)
    aff = aff.at[0:cout, 18].set(scale).at[0:cout, 19].set(beta)
    return aff


def kernel(xyz, points, conv0_w, conv0_b, bn0_g, bn0_b, conv1_w, conv1_b,
           bn1_g, bn1_b, wn0_w, wn0_b, wnbn0_g, wnbn0_b, wn1_w, wn1_b,
           wnbn1_g, wnbn1_b, wn2_w, wn2_b, wnbn2_g, wnbn2_b,
           lin_w, lin_b, bnl_g, bnl_b):
    keys, xyzp = _run_keys(xyz)
    idx, g8, hist, macc = _run_select(keys, xyzp)

    X2 = _run_e(points, hist, conv0_w, conv0_b, bn0_g, bn0_b,
                conv1_w, conv1_b, bn1_g, bn1_b)

    mac = jnp.sum(macc, axis=0)
    Pf = jnp.float32(_P)
    meanG = jnp.sum(mac[0:3, :], axis=1) / Pf
    xx, xy, xz, yy, yz, zz = [jnp.sum(mac[i]) / Pf for i in range(3, 9)]
    MG = jnp.stack([jnp.stack([xx, xy, xz]),
                    jnp.stack([xy, yy, yz]),
                    jnp.stack([xz, yz, zz])])
    aff0 = _pack_wn(8, meanG, MG, wn0_w, wn0_b, wnbn0_g, wnbn0_b)

    acc1 = _run_km(1, g8, aff0, aff0)
    MU1 = acc1[:, 0:8] / Pf
    mU1 = acc1[:, 8] / Pf
    aff1 = _pack_wn(8, mU1, MU1, wn1_w, wn1_b, wnbn1_g, wnbn1_b)

    acc2 = _run_km(2, g8, aff0, aff1)
    MU2 = acc2[:, 0:8] / Pf
    mU2 = acc2[:, 8] / Pf
    aff2 = _pack_wn(16, mU2, MU2, wn2_w, wn2_b, wnbn2_g, wnbn2_b)

    ag = _run_gather(X2, idx.reshape(_P))

    out_pre, accF = _run_kwf(g8, ag, aff0, aff1, aff2,
                             jnp.transpose(lin_w, (1, 0)).astype(jnp.bfloat16),
                             lin_b[None, :])
    out = _run_f(out_pre, accF, bnl_g, bnl_b)
    return (xyz, out)
